# baseline (speedup 1.0000x reference)
import math

import jax
import jax.numpy as jnp
from jax.experimental import pallas as pl

D_MODEL = 128
NUM_CLASSES = 32
ETA = 0.5
RC = 5.0
NR = 4
BATCH = 4096
NEIGH = 50

_C0 = 0.28209479177387814
_C1 = 0.4886025119029199
_RS = (0.0, 1.25, 2.5, 3.75)

_BB = 128


def _pe_kernel(x_ref, o_ref):
    x = x_ref[...]
    px = x[..., 0]
    py = x[..., 1]
    pz = x[..., 2]
    cls = x[..., 3]

    r2 = px * px + py * py + pz * pz
    r = jnp.sqrt(r2)
    r_safe = jnp.where(r == 0.0, 1.0, r)
    inv_r = 1.0 / r_safe

    ang0 = jnp.full(r.shape, _C0, dtype=x.dtype)
    ang1 = _C1 * py * inv_r
    ang2 = _C1 * pz * inv_r
    ang3 = _C1 * px * inv_r

    r_clip = jnp.minimum(r, RC)
    env = 0.5 * jnp.cos((math.pi / RC) * r_clip) + 0.5

    valid = (r != 0.0).astype(x.dtype)
    enva = env * valid

    p0 = ang0 * jnp.exp(-ETA * (_RS[0] - r) ** 2) * enva
    p1 = ang1 * jnp.exp(-ETA * (_RS[1] - r) ** 2) * enva
    p2 = ang2 * jnp.exp(-ETA * (_RS[2] - r) ** 2) * enva
    p3 = ang3 * jnp.exp(-ETA * (_RS[3] - r) ** 2) * enva

    lane = jax.lax.broadcasted_iota(jnp.int32, (1, 1, D_MODEL), 2)
    nr_idx = lane // NUM_CLASSES
    cmask = (lane % NUM_CLASSES).astype(x.dtype) == cls[..., None]

    psel = jnp.where(
        nr_idx == 0, p0[..., None],
        jnp.where(nr_idx == 1, p1[..., None],
                  jnp.where(nr_idx == 2, p2[..., None], p3[..., None])))
    pos = jnp.where(cmask, psel, 0.0)

    nrm = jnp.sqrt(jnp.sum(pos * pos, axis=1, keepdims=True))
    pos = pos / jnp.maximum(nrm, 1e-12)

    mean = jnp.mean(pos, axis=-1, keepdims=True)
    d = pos - mean
    var = jnp.sum(d * d, axis=-1, keepdims=True) * (1.0 / (D_MODEL - 1))
    o_ref[...] = d / (jnp.sqrt(var) + 1e-6)


def kernel(x):
    B, N, _ = x.shape
    grid = (B // _BB,)
    return pl.pallas_call(
        _pe_kernel,
        grid=grid,
        in_specs=[pl.BlockSpec((_BB, N, 4), lambda i: (i, 0, 0))],
        out_specs=pl.BlockSpec((_BB, N, D_MODEL), lambda i: (i, 0, 0)),
        out_shape=jax.ShapeDtypeStruct((B, N, D_MODEL), x.dtype),
    )(x)
